# Initial kernel scaffold; baseline (speedup 1.0000x reference)
import jax
import jax.numpy as jnp
from jax.experimental import pallas as pl


def kernel(img):
    raise NotImplementedError("write your pallas kernel here")


# speedup vs baseline: 1.0006x; 1.0006x over previous
import functools

import jax
import jax.numpy as jnp
from jax.experimental import pallas as pl
from jax.experimental.pallas import tpu as pltpu

_MIN_DISABLED = 32
_MAX_DISABLED = 128


def _mask_kernel(scalars_ref, img_ref, out_ref):
    start = scalars_ref[0]
    count = scalars_ref[1]
    cols = jax.lax.broadcasted_iota(jnp.int32, img_ref.shape, 1)
    d = cols - start
    wrapped = jnp.where(d < 0, d + img_ref.shape[1], d)
    keep = wrapped >= count
    out_ref[...] = jnp.where(keep, img_ref[...], jnp.float32(0.0))


def kernel(img):
    rows, tof_count = img.shape
    key = jax.random.key(42)
    k1, k2 = jax.random.split(key)
    count = jax.random.randint(k1, (), _MIN_DISABLED, _MAX_DISABLED + 1)
    start = jax.random.randint(k2, (), 0, tof_count)
    scalars = jnp.stack([start.astype(jnp.int32), count.astype(jnp.int32)])

    block_rows = 1024
    grid = (rows // block_rows,)
    return pl.pallas_call(
        _mask_kernel,
        grid_spec=pltpu.PrefetchScalarGridSpec(
            num_scalar_prefetch=1,
            grid=grid,
            in_specs=[pl.BlockSpec((block_rows, tof_count), lambda i, s: (i, 0))],
            out_specs=pl.BlockSpec((block_rows, tof_count), lambda i, s: (i, 0)),
        ),
        out_shape=jax.ShapeDtypeStruct((rows, tof_count), jnp.float32),
    )(scalars, img)
